# Initial kernel scaffold; baseline (speedup 1.0000x reference)
import jax
import jax.numpy as jnp
from jax.experimental import pallas as pl


def kernel(x, edge_index, TRAIN, Wc0, bc0, Wc1, bc1, Wc2, bc2, Wf1, bf1, Wf2, bf2, Wf3, bf3):
    raise NotImplementedError("write your pallas kernel here")


# speedup vs baseline: 12.6016x; 12.6016x over previous
import jax
import jax.numpy as jnp
from jax import lax
from jax.experimental import pallas as pl
from jax.experimental.pallas import tpu as pltpu
from jax.experimental.pallas import tpu_sc as plsc

N = 10000
E = 160000
H = 256
DQ = 64
NC = 2
NS = 16
ROWS_A = 640
ROWS_B = 400
CB = 125
RPS = 80
RPT = 40
NB = 5
ZB = 80
DZB = 40

_mesh = plsc.VectorSubcoreMesh(core_axis_name="c", subcore_axis_name="s")


def _deg_body(dst2, ones_hbm, zer16_hbm, out, didx, ones_v, zer_v, deg_sh):
    c = lax.axis_index("c")
    s = lax.axis_index("s")
    t = s * NC + c
    pltpu.sync_copy(ones_hbm, ones_v)
    pltpu.sync_copy(zer16_hbm, zer_v)
    r0 = s * ROWS_A
    nz = jnp.where(s == NS - 1, ROWS_B // DZB, ROWS_A // DZB)

    def zb(j, carry):
        pltpu.sync_copy(zer_v, deg_sh.at[pl.ds(r0 + j * DZB, DZB)])
        return carry

    lax.fori_loop(0, nz, zb, 0)
    pltpu.sync_copy(dst2.at[pl.ds(t * RPT, RPT)], didx)
    plsc.subcore_barrier()

    def body(i, carry):
        pltpu.sync_copy(ones_v, deg_sh.at[didx.at[i]], add=True)
        return carry

    lax.fori_loop(0, RPT, body, 0)
    plsc.subcore_barrier()

    @pl.when(s < NS - 1)
    def _():
        pltpu.sync_copy(deg_sh.at[pl.ds(r0, ROWS_A)],
                        out.at[c, pl.ds(r0, ROWS_A)])

    @pl.when(s == NS - 1)
    def _():
        pltpu.sync_copy(deg_sh.at[pl.ds((NS - 1) * ROWS_A, ROWS_B)],
                        out.at[c, pl.ds((NS - 1) * ROWS_A, ROWS_B)])


def _deg_call(dst2, ones16, zer16):
    return pl.kernel(
        _deg_body,
        out_type=jax.ShapeDtypeStruct((NC, N, 16), jnp.float32),
        mesh=_mesh,
        compiler_params=pltpu.CompilerParams(use_tc_tiling_on_sc=False),
        scratch_types=[
            pltpu.VMEM((RPT, CB), jnp.int32),
            pltpu.VMEM((CB, 16), jnp.float32),
            pltpu.VMEM((DZB, 16), jnp.float32),
            pltpu.VMEM_SHARED((N, 16), jnp.float32),
        ],
    )(dst2, ones16, zer16)


def _agg_body(zT, src2, dst2, zrow_hbm, out, sidx, didx, zer_v,
              b0, b1, b2, b3, b4, gsem, ssem, acc_sh):
    c = lax.axis_index("c")
    s = lax.axis_index("s")
    rows = [b0, b1, b2, b3, b4]
    r0 = s * ROWS_A
    pltpu.sync_copy(zrow_hbm, zer_v)
    pltpu.sync_copy(src2.at[pl.ds(s * RPS, RPS)], sidx)
    pltpu.sync_copy(dst2.at[pl.ds(s * RPS, RPS)], didx)
    nz = jnp.where(s == NS - 1, ROWS_B // ZB, ROWS_A // ZB)

    for k in range(2):
        q = 2 * k + c
        zc = zT.at[q]

        def zb(j, carry):
            pltpu.sync_copy(zer_v, acc_sh.at[pl.ds(r0 + j * ZB, ZB)])
            return carry

        lax.fori_loop(0, nz, zb, 0)
        plsc.subcore_barrier()

        for b in range(NB):
            pltpu.async_copy(zc.at[sidx.at[b]], rows[b], gsem.at[b])

        def group(g, carry):
            for b in range(NB):
                i = g * NB + b
                pltpu.make_async_copy(zc.at[sidx.at[i]], rows[b],
                                      gsem.at[b]).wait()
                pltpu.async_copy(rows[b], acc_sh.at[didx.at[i]], ssem.at[b],
                                 add=True)
            for b in range(NB):
                i = g * NB + b
                pltpu.make_async_copy(rows[b], acc_sh.at[didx.at[i]],
                                      ssem.at[b]).wait()
                pltpu.async_copy(zc.at[sidx.at[i + NB]], rows[b], gsem.at[b])
            return carry

        lax.fori_loop(0, RPS // NB - 1, group, 0)
        for b in range(NB):
            i = (RPS // NB - 1) * NB + b
            pltpu.make_async_copy(zc.at[sidx.at[i]], rows[b],
                                  gsem.at[b]).wait()
            pltpu.async_copy(rows[b], acc_sh.at[didx.at[i]], ssem.at[b],
                             add=True)
        for b in range(NB):
            i = (RPS // NB - 1) * NB + b
            pltpu.make_async_copy(rows[b], acc_sh.at[didx.at[i]],
                                  ssem.at[b]).wait()
        plsc.subcore_barrier()

        @pl.when(s < NS - 1)
        def _():
            pltpu.sync_copy(acc_sh.at[pl.ds(r0, ROWS_A)],
                            out.at[q, pl.ds(r0, ROWS_A)])

        @pl.when(s == NS - 1)
        def _():
            pltpu.sync_copy(acc_sh.at[pl.ds((NS - 1) * ROWS_A, ROWS_B)],
                            out.at[q, pl.ds((NS - 1) * ROWS_A, ROWS_B)])


def _agg_call(zT, src2, dst2, zrow):
    return pl.kernel(
        _agg_body,
        out_type=jax.ShapeDtypeStruct((4, N, DQ), jnp.float32),
        mesh=_mesh,
        compiler_params=pltpu.CompilerParams(use_tc_tiling_on_sc=False),
        scratch_types=[
            pltpu.VMEM((RPS, CB), jnp.int32),
            pltpu.VMEM((RPS, CB), jnp.int32),
            pltpu.VMEM((ZB, DQ), jnp.float32),
            pltpu.VMEM((CB, DQ), jnp.float32),
            pltpu.VMEM((CB, DQ), jnp.float32),
            pltpu.VMEM((CB, DQ), jnp.float32),
            pltpu.VMEM((CB, DQ), jnp.float32),
            pltpu.VMEM((CB, DQ), jnp.float32),
            pltpu.SemaphoreType.DMA((NB,)),
            pltpu.SemaphoreType.DMA((NB,)),
            pltpu.VMEM_SHARED((N, DQ), jnp.float32),
        ],
    )(zT, src2, dst2, zrow)


_BM = 1000


def _dinv_of(degp_ref):
    return lax.rsqrt(1.0 + degp_ref[0, :, 0:1] + degp_ref[1, :, 0:1])


def _dot(a, b):
    return jnp.dot(a, b, preferred_element_type=jnp.float32,
                   precision=lax.Precision.HIGHEST)


def _split_q(z, o_ref):
    for k in range(4):
        o_ref[k] = z[:, k * DQ:(k + 1) * DQ]


def _mm_scale_body(x_ref, w_ref, degp_ref, o_ref):
    dinv = _dinv_of(degp_ref)
    z = _dot(x_ref[...], w_ref[...]) * dinv
    _split_q(z, o_ref)


def _mm_scale(x, W, degp):
    return pl.pallas_call(
        _mm_scale_body,
        grid=(N // _BM,),
        in_specs=[
            pl.BlockSpec((_BM, H), lambda i: (i, 0)),
            pl.BlockSpec((H, H), lambda i: (0, 0)),
            pl.BlockSpec((2, _BM, 16), lambda i: (0, i, 0)),
        ],
        out_specs=pl.BlockSpec((4, _BM, DQ), lambda i: (0, i, 0)),
        out_shape=jax.ShapeDtypeStruct((4, N, DQ), jnp.float32),
    )(x, W, degp)


def _combine_body(has_res, refs):
    if has_res:
        agg_ref, z_ref, degp_ref, b_ref, res_ref, w_ref, y_ref, zo_ref = refs
    else:
        agg_ref, z_ref, degp_ref, b_ref, w_ref, y_ref, zo_ref = refs
    dinv = _dinv_of(degp_ref)
    aggz = jnp.concatenate(
        [agg_ref[k] + z_ref[k] for k in range(4)], axis=1)
    y = aggz * dinv + b_ref[...]
    if has_res:
        y = y + res_ref[...]
    y = jnp.maximum(y, 0.0)
    y_ref[...] = y
    zn = _dot(y, w_ref[...]) * dinv
    _split_q(zn, zo_ref)


def _combine_mm(agg, z, degp, bias, res, Wn):
    has_res = res is not None
    in_specs = [
        pl.BlockSpec((4, _BM, DQ), lambda i: (0, i, 0)),
        pl.BlockSpec((4, _BM, DQ), lambda i: (0, i, 0)),
        pl.BlockSpec((2, _BM, 16), lambda i: (0, i, 0)),
        pl.BlockSpec((1, H), lambda i: (0, 0)),
    ]
    args = [agg, z, degp, bias]
    if has_res:
        in_specs.append(pl.BlockSpec((_BM, H), lambda i: (i, 0)))
        args.append(res)
    in_specs.append(pl.BlockSpec((H, H), lambda i: (0, 0)))
    args.append(Wn)
    return pl.pallas_call(
        lambda *refs: _combine_body(has_res, refs),
        grid=(N // _BM,),
        in_specs=in_specs,
        out_specs=[
            pl.BlockSpec((_BM, H), lambda i: (i, 0)),
            pl.BlockSpec((4, _BM, DQ), lambda i: (0, i, 0)),
        ],
        out_shape=[
            jax.ShapeDtypeStruct((N, H), jnp.float32),
            jax.ShapeDtypeStruct((4, N, DQ), jnp.float32),
        ],
    )(*args)


def _final_body(agg_ref, z_ref, degp_ref, b_ref, res_ref, w1_ref, b1_ref,
                w2_ref, b2_ref, w3_ref, b3_ref, o_ref):
    dinv = _dinv_of(degp_ref)
    aggz = jnp.concatenate(
        [agg_ref[k] + z_ref[k] for k in range(4)], axis=1)
    y = jnp.maximum(aggz * dinv + b_ref[...] + res_ref[...], 0.0)
    h = jnp.maximum(_dot(y, w1_ref[...]) + b1_ref[...], 0.0)
    h = jnp.maximum(_dot(h, w2_ref[...]) + b2_ref[...], 0.0)
    o = _dot(h, w3_ref[...]) + b3_ref[...]
    m = jnp.max(o, axis=1, keepdims=True)
    e = jnp.exp(o - m)
    o_ref[...] = (o - m) - jnp.log(jnp.sum(e, axis=1, keepdims=True))


def _final(agg, z, degp, bias, res, W1, b1, W2, b2, W3, b3):
    nc = W3.shape[1]
    return pl.pallas_call(
        _final_body,
        grid=(N // _BM,),
        in_specs=[
            pl.BlockSpec((4, _BM, DQ), lambda i: (0, i, 0)),
            pl.BlockSpec((4, _BM, DQ), lambda i: (0, i, 0)),
            pl.BlockSpec((2, _BM, 16), lambda i: (0, i, 0)),
            pl.BlockSpec((1, H), lambda i: (0, 0)),
            pl.BlockSpec((_BM, H), lambda i: (i, 0)),
            pl.BlockSpec((H, H), lambda i: (0, 0)),
            pl.BlockSpec((1, H), lambda i: (0, 0)),
            pl.BlockSpec((H, H), lambda i: (0, 0)),
            pl.BlockSpec((1, H), lambda i: (0, 0)),
            pl.BlockSpec((H, nc), lambda i: (0, 0)),
            pl.BlockSpec((1, nc), lambda i: (0, 0)),
        ],
        out_specs=pl.BlockSpec((_BM, nc), lambda i: (i, 0)),
        out_shape=jax.ShapeDtypeStruct((N, nc), jnp.float32),
    )(agg, z, degp, bias, res, W1, b1, W2, b2, W3, b3)


def kernel(x, edge_index, TRAIN, Wc0, bc0, Wc1, bc1, Wc2, bc2,
           Wf1, bf1, Wf2, bf2, Wf3, bf3):
    del TRAIN
    src2 = edge_index[0].reshape(E // CB, CB)
    dst2 = edge_index[1].reshape(E // CB, CB)
    ones16 = jnp.ones((CB, 16), jnp.float32)
    zer16 = jnp.zeros((DZB, 16), jnp.float32)
    zrow = jnp.zeros((ZB, DQ), jnp.float32)

    degp = _deg_call(dst2, ones16, zer16)
    z0 = _mm_scale(x, Wc0, degp)
    agg0 = _agg_call(z0, src2, dst2, zrow)
    y1, z1 = _combine_mm(agg0, z0, degp, bc0.reshape(1, H), None, Wc1)
    agg1 = _agg_call(z1, src2, dst2, zrow)
    _, z2 = _combine_mm(agg1, z1, degp, bc1.reshape(1, H), y1, Wc2)
    agg2 = _agg_call(z2, src2, dst2, zrow)
    return _final(agg2, z2, degp, bc2.reshape(1, H), y1,
                  Wf1, bf1.reshape(1, H), Wf2, bf2.reshape(1, H),
                  Wf3, bf3.reshape(1, 7))


# speedup vs baseline: 12.9237x; 1.0256x over previous
import jax
import jax.numpy as jnp
from jax import lax
from jax.experimental import pallas as pl
from jax.experimental.pallas import tpu as pltpu
from jax.experimental.pallas import tpu_sc as plsc

N = 10000
E = 160000
H = 256
DQ = 64
NC = 2
NS = 16
ROWS_A = 640
ROWS_B = 400
CB = 125
RPS = 80
RPT = 40
NB = 8
ZB = 80
DZB = 40

_mesh = plsc.VectorSubcoreMesh(core_axis_name="c", subcore_axis_name="s")


def _deg_body(dst2, ones_hbm, zer16_hbm, out, didx, ones_v, zer_v, deg_sh):
    c = lax.axis_index("c")
    s = lax.axis_index("s")
    t = s * NC + c
    pltpu.sync_copy(ones_hbm, ones_v)
    pltpu.sync_copy(zer16_hbm, zer_v)
    r0 = s * ROWS_A
    nz = jnp.where(s == NS - 1, ROWS_B // DZB, ROWS_A // DZB)

    def zb(j, carry):
        pltpu.sync_copy(zer_v, deg_sh.at[pl.ds(r0 + j * DZB, DZB)])
        return carry

    lax.fori_loop(0, nz, zb, 0)
    pltpu.sync_copy(dst2.at[pl.ds(t * RPT, RPT)], didx)
    plsc.subcore_barrier()

    def body(i, carry):
        pltpu.sync_copy(ones_v, deg_sh.at[didx.at[i]], add=True)
        return carry

    lax.fori_loop(0, RPT, body, 0)
    plsc.subcore_barrier()

    @pl.when(s < NS - 1)
    def _():
        pltpu.sync_copy(deg_sh.at[pl.ds(r0, ROWS_A)],
                        out.at[c, pl.ds(r0, ROWS_A)])

    @pl.when(s == NS - 1)
    def _():
        pltpu.sync_copy(deg_sh.at[pl.ds((NS - 1) * ROWS_A, ROWS_B)],
                        out.at[c, pl.ds((NS - 1) * ROWS_A, ROWS_B)])


def _deg_call(dst2, ones16, zer16):
    return pl.kernel(
        _deg_body,
        out_type=jax.ShapeDtypeStruct((NC, N, 16), jnp.float32),
        mesh=_mesh,
        compiler_params=pltpu.CompilerParams(use_tc_tiling_on_sc=False),
        scratch_types=[
            pltpu.VMEM((RPT, CB), jnp.int32),
            pltpu.VMEM((CB, 16), jnp.float32),
            pltpu.VMEM((DZB, 16), jnp.float32),
            pltpu.VMEM_SHARED((N, 16), jnp.float32),
        ],
    )(dst2, ones16, zer16)


def _agg_body(zT, src2, dst2, zrow_hbm, out, sidx, didx, zer_v,
              b0, b1, b2, b3, b4, b5, b6, b7, gsem, ssem, acc_sh):
    c = lax.axis_index("c")
    s = lax.axis_index("s")
    rows = [b0, b1, b2, b3, b4, b5, b6, b7]
    r0 = s * ROWS_A
    pltpu.sync_copy(zrow_hbm, zer_v)
    pltpu.sync_copy(src2.at[pl.ds(s * RPS, RPS)], sidx)
    pltpu.sync_copy(dst2.at[pl.ds(s * RPS, RPS)], didx)
    nz = jnp.where(s == NS - 1, ROWS_B // ZB, ROWS_A // ZB)

    for k in range(2):
        q = 2 * k + c
        zc = zT.at[q]

        def zb(j, carry):
            pltpu.sync_copy(zer_v, acc_sh.at[pl.ds(r0 + j * ZB, ZB)])
            return carry

        lax.fori_loop(0, nz, zb, 0)
        plsc.subcore_barrier()

        for b in range(NB):
            pltpu.async_copy(zc.at[sidx.at[b]], rows[b], gsem.at[b])

        def group(g, carry):
            for b in range(NB):
                i = g * NB + b
                pltpu.make_async_copy(zc.at[sidx.at[i]], rows[b],
                                      gsem.at[b]).wait()
                pltpu.async_copy(rows[b], acc_sh.at[didx.at[i]], ssem.at[b],
                                 add=True)
            for b in range(NB):
                i = g * NB + b
                pltpu.make_async_copy(rows[b], acc_sh.at[didx.at[i]],
                                      ssem.at[b]).wait()
                pltpu.async_copy(zc.at[sidx.at[i + NB]], rows[b], gsem.at[b])
            return carry

        lax.fori_loop(0, RPS // NB - 1, group, 0)
        for b in range(NB):
            i = (RPS // NB - 1) * NB + b
            pltpu.make_async_copy(zc.at[sidx.at[i]], rows[b],
                                  gsem.at[b]).wait()
            pltpu.async_copy(rows[b], acc_sh.at[didx.at[i]], ssem.at[b],
                             add=True)
        for b in range(NB):
            i = (RPS // NB - 1) * NB + b
            pltpu.make_async_copy(rows[b], acc_sh.at[didx.at[i]],
                                  ssem.at[b]).wait()
        plsc.subcore_barrier()

        @pl.when(s < NS - 1)
        def _():
            pltpu.sync_copy(acc_sh.at[pl.ds(r0, ROWS_A)],
                            out.at[q, pl.ds(r0, ROWS_A)])

        @pl.when(s == NS - 1)
        def _():
            pltpu.sync_copy(acc_sh.at[pl.ds((NS - 1) * ROWS_A, ROWS_B)],
                            out.at[q, pl.ds((NS - 1) * ROWS_A, ROWS_B)])


def _agg_call(zT, src2, dst2, zrow):
    return pl.kernel(
        _agg_body,
        out_type=jax.ShapeDtypeStruct((4, N, DQ), jnp.float32),
        mesh=_mesh,
        compiler_params=pltpu.CompilerParams(use_tc_tiling_on_sc=False),
        scratch_types=[
            pltpu.VMEM((RPS, CB), jnp.int32),
            pltpu.VMEM((RPS, CB), jnp.int32),
            pltpu.VMEM((ZB, DQ), jnp.float32),
            pltpu.VMEM((CB, DQ), jnp.float32),
            pltpu.VMEM((CB, DQ), jnp.float32),
            pltpu.VMEM((CB, DQ), jnp.float32),
            pltpu.VMEM((CB, DQ), jnp.float32),
            pltpu.VMEM((CB, DQ), jnp.float32),
            pltpu.VMEM((CB, DQ), jnp.float32),
            pltpu.VMEM((CB, DQ), jnp.float32),
            pltpu.VMEM((CB, DQ), jnp.float32),
            pltpu.SemaphoreType.DMA((NB,)),
            pltpu.SemaphoreType.DMA((NB,)),
            pltpu.VMEM_SHARED((N, DQ), jnp.float32),
        ],
    )(zT, src2, dst2, zrow)


_BM = 1000


def _dinv_of(degp_ref):
    return lax.rsqrt(1.0 + degp_ref[0, :, 0:1] + degp_ref[1, :, 0:1])


def _dot(a, b):
    return jnp.dot(a, b, preferred_element_type=jnp.float32,
                   precision=lax.Precision.HIGHEST)


def _split_q(z, o_ref):
    for k in range(4):
        o_ref[k] = z[:, k * DQ:(k + 1) * DQ]


def _mm_scale_body(x_ref, w_ref, degp_ref, o_ref):
    dinv = _dinv_of(degp_ref)
    z = _dot(x_ref[...], w_ref[...]) * dinv
    _split_q(z, o_ref)


def _mm_scale(x, W, degp):
    return pl.pallas_call(
        _mm_scale_body,
        grid=(N // _BM,),
        in_specs=[
            pl.BlockSpec((_BM, H), lambda i: (i, 0)),
            pl.BlockSpec((H, H), lambda i: (0, 0)),
            pl.BlockSpec((2, _BM, 16), lambda i: (0, i, 0)),
        ],
        out_specs=pl.BlockSpec((4, _BM, DQ), lambda i: (0, i, 0)),
        out_shape=jax.ShapeDtypeStruct((4, N, DQ), jnp.float32),
    )(x, W, degp)


def _combine_body(has_res, refs):
    if has_res:
        agg_ref, z_ref, degp_ref, b_ref, res_ref, w_ref, y_ref, zo_ref = refs
    else:
        agg_ref, z_ref, degp_ref, b_ref, w_ref, y_ref, zo_ref = refs
    dinv = _dinv_of(degp_ref)
    aggz = jnp.concatenate(
        [agg_ref[k] + z_ref[k] for k in range(4)], axis=1)
    y = aggz * dinv + b_ref[...]
    if has_res:
        y = y + res_ref[...]
    y = jnp.maximum(y, 0.0)
    y_ref[...] = y
    zn = _dot(y, w_ref[...]) * dinv
    _split_q(zn, zo_ref)


def _combine_mm(agg, z, degp, bias, res, Wn):
    has_res = res is not None
    in_specs = [
        pl.BlockSpec((4, _BM, DQ), lambda i: (0, i, 0)),
        pl.BlockSpec((4, _BM, DQ), lambda i: (0, i, 0)),
        pl.BlockSpec((2, _BM, 16), lambda i: (0, i, 0)),
        pl.BlockSpec((1, H), lambda i: (0, 0)),
    ]
    args = [agg, z, degp, bias]
    if has_res:
        in_specs.append(pl.BlockSpec((_BM, H), lambda i: (i, 0)))
        args.append(res)
    in_specs.append(pl.BlockSpec((H, H), lambda i: (0, 0)))
    args.append(Wn)
    return pl.pallas_call(
        lambda *refs: _combine_body(has_res, refs),
        grid=(N // _BM,),
        in_specs=in_specs,
        out_specs=[
            pl.BlockSpec((_BM, H), lambda i: (i, 0)),
            pl.BlockSpec((4, _BM, DQ), lambda i: (0, i, 0)),
        ],
        out_shape=[
            jax.ShapeDtypeStruct((N, H), jnp.float32),
            jax.ShapeDtypeStruct((4, N, DQ), jnp.float32),
        ],
    )(*args)


def _final_body(agg_ref, z_ref, degp_ref, b_ref, res_ref, w1_ref, b1_ref,
                w2_ref, b2_ref, w3_ref, b3_ref, o_ref):
    dinv = _dinv_of(degp_ref)
    aggz = jnp.concatenate(
        [agg_ref[k] + z_ref[k] for k in range(4)], axis=1)
    y = jnp.maximum(aggz * dinv + b_ref[...] + res_ref[...], 0.0)
    h = jnp.maximum(_dot(y, w1_ref[...]) + b1_ref[...], 0.0)
    h = jnp.maximum(_dot(h, w2_ref[...]) + b2_ref[...], 0.0)
    o = _dot(h, w3_ref[...]) + b3_ref[...]
    m = jnp.max(o, axis=1, keepdims=True)
    e = jnp.exp(o - m)
    o_ref[...] = (o - m) - jnp.log(jnp.sum(e, axis=1, keepdims=True))


def _final(agg, z, degp, bias, res, W1, b1, W2, b2, W3, b3):
    nc = W3.shape[1]
    return pl.pallas_call(
        _final_body,
        grid=(N // _BM,),
        in_specs=[
            pl.BlockSpec((4, _BM, DQ), lambda i: (0, i, 0)),
            pl.BlockSpec((4, _BM, DQ), lambda i: (0, i, 0)),
            pl.BlockSpec((2, _BM, 16), lambda i: (0, i, 0)),
            pl.BlockSpec((1, H), lambda i: (0, 0)),
            pl.BlockSpec((_BM, H), lambda i: (i, 0)),
            pl.BlockSpec((H, H), lambda i: (0, 0)),
            pl.BlockSpec((1, H), lambda i: (0, 0)),
            pl.BlockSpec((H, H), lambda i: (0, 0)),
            pl.BlockSpec((1, H), lambda i: (0, 0)),
            pl.BlockSpec((H, nc), lambda i: (0, 0)),
            pl.BlockSpec((1, nc), lambda i: (0, 0)),
        ],
        out_specs=pl.BlockSpec((_BM, nc), lambda i: (i, 0)),
        out_shape=jax.ShapeDtypeStruct((N, nc), jnp.float32),
    )(agg, z, degp, bias, res, W1, b1, W2, b2, W3, b3)


def kernel(x, edge_index, TRAIN, Wc0, bc0, Wc1, bc1, Wc2, bc2,
           Wf1, bf1, Wf2, bf2, Wf3, bf3):
    del TRAIN
    src2 = edge_index[0].reshape(E // CB, CB)
    dst2 = edge_index[1].reshape(E // CB, CB)
    ones16 = jnp.ones((CB, 16), jnp.float32)
    zer16 = jnp.zeros((DZB, 16), jnp.float32)
    zrow = jnp.zeros((ZB, DQ), jnp.float32)

    degp = _deg_call(dst2, ones16, zer16)
    z0 = _mm_scale(x, Wc0, degp)
    agg0 = _agg_call(z0, src2, dst2, zrow)
    y1, z1 = _combine_mm(agg0, z0, degp, bc0.reshape(1, H), None, Wc1)
    agg1 = _agg_call(z1, src2, dst2, zrow)
    _, z2 = _combine_mm(agg1, z1, degp, bc1.reshape(1, H), y1, Wc2)
    agg2 = _agg_call(z2, src2, dst2, zrow)
    return _final(agg2, z2, degp, bc2.reshape(1, H), y1,
                  Wf1, bf1.reshape(1, H), Wf2, bf2.reshape(1, H),
                  Wf3, bf3.reshape(1, 7))


# speedup vs baseline: 13.5883x; 1.0514x over previous
import jax
import jax.numpy as jnp
from jax import lax
from jax.experimental import pallas as pl
from jax.experimental.pallas import tpu as pltpu
from jax.experimental.pallas import tpu_sc as plsc

N = 10000
E = 160000
H = 256
DQ = 64
NC = 2
NS = 16
ROWS_A = 640
ROWS_B = 400
CB = 125
RPS = 80
RPT = 40
NB = 8
ZB = 80
DZB = 40

_mesh = plsc.VectorSubcoreMesh(core_axis_name="c", subcore_axis_name="s")


def _deg_body(dst2, ones_hbm, zer16_hbm, out, didx, ones_v, zer_v, deg_sh):
    c = lax.axis_index("c")
    s = lax.axis_index("s")
    t = s * NC + c
    pltpu.sync_copy(ones_hbm, ones_v)
    pltpu.sync_copy(zer16_hbm, zer_v)
    r0 = s * ROWS_A
    nz = jnp.where(s == NS - 1, ROWS_B // DZB, ROWS_A // DZB)

    def zb(j, carry):
        pltpu.sync_copy(zer_v, deg_sh.at[pl.ds(r0 + j * DZB, DZB)])
        return carry

    lax.fori_loop(0, nz, zb, 0)
    pltpu.sync_copy(dst2.at[pl.ds(t * RPT, RPT)], didx)
    plsc.subcore_barrier()

    def body(i, carry):
        pltpu.sync_copy(ones_v, deg_sh.at[didx.at[i]], add=True)
        return carry

    lax.fori_loop(0, RPT, body, 0)
    plsc.subcore_barrier()

    @pl.when(s < NS - 1)
    def _():
        pltpu.sync_copy(deg_sh.at[pl.ds(r0, ROWS_A)],
                        out.at[c, pl.ds(r0, ROWS_A)])

    @pl.when(s == NS - 1)
    def _():
        pltpu.sync_copy(deg_sh.at[pl.ds((NS - 1) * ROWS_A, ROWS_B)],
                        out.at[c, pl.ds((NS - 1) * ROWS_A, ROWS_B)])


def _deg_call(dst2, ones16, zer16):
    return pl.kernel(
        _deg_body,
        out_type=jax.ShapeDtypeStruct((NC, N, 16), jnp.float32),
        mesh=_mesh,
        compiler_params=pltpu.CompilerParams(use_tc_tiling_on_sc=False),
        scratch_types=[
            pltpu.VMEM((RPT, CB), jnp.int32),
            pltpu.VMEM((CB, 16), jnp.float32),
            pltpu.VMEM((DZB, 16), jnp.float32),
            pltpu.VMEM_SHARED((N, 16), jnp.float32),
        ],
    )(dst2, ones16, zer16)


def _agg_body(zT, src2, dst2, zrow_hbm, out, sidx, didx, zer_v,
              b0, b1, b2, b3, b4, b5, b6, b7, gsem, ssem, acc_sh):
    c = lax.axis_index("c")
    s = lax.axis_index("s")
    rows = [b0, b1, b2, b3, b4, b5, b6, b7]
    r0 = s * ROWS_A
    pltpu.sync_copy(zrow_hbm, zer_v)
    pltpu.sync_copy(src2.at[pl.ds(s * RPS, RPS)], sidx)
    pltpu.sync_copy(dst2.at[pl.ds(s * RPS, RPS)], didx)
    nz = jnp.where(s == NS - 1, ROWS_B // ZB, ROWS_A // ZB)

    for k in range(2):
        q = 2 * k + c
        zc = zT.at[q]

        def zb(j, carry):
            pltpu.sync_copy(zer_v, acc_sh.at[pl.ds(r0 + j * ZB, ZB)])
            return carry

        lax.fori_loop(0, nz, zb, 0)
        plsc.subcore_barrier()

        for b in range(NB):
            pltpu.async_copy(zc.at[sidx.at[b]], rows[b], gsem.at[b])

        def group(g, carry):
            for b in range(NB):
                i = g * NB + b
                pltpu.make_async_copy(zc.at[sidx.at[i]], rows[b],
                                      gsem.at[b]).wait()
                pltpu.async_copy(rows[b], acc_sh.at[didx.at[i]], ssem.at[b],
                                 add=True)
            for b in range(NB):
                i = g * NB + b
                pltpu.make_async_copy(rows[b], acc_sh.at[didx.at[i]],
                                      ssem.at[b]).wait()
                pltpu.async_copy(zc.at[sidx.at[i + NB]], rows[b], gsem.at[b])
            return carry

        lax.fori_loop(0, RPS // NB - 1, group, 0)
        for b in range(NB):
            i = (RPS // NB - 1) * NB + b
            pltpu.make_async_copy(zc.at[sidx.at[i]], rows[b],
                                  gsem.at[b]).wait()
            pltpu.async_copy(rows[b], acc_sh.at[didx.at[i]], ssem.at[b],
                             add=True)
        for b in range(NB):
            i = (RPS // NB - 1) * NB + b
            pltpu.make_async_copy(rows[b], acc_sh.at[didx.at[i]],
                                  ssem.at[b]).wait()
        plsc.subcore_barrier()

        @pl.when(s < NS - 1)
        def _():
            pltpu.sync_copy(acc_sh.at[pl.ds(r0, ROWS_A)],
                            out.at[q, pl.ds(r0, ROWS_A)])

        @pl.when(s == NS - 1)
        def _():
            pltpu.sync_copy(acc_sh.at[pl.ds((NS - 1) * ROWS_A, ROWS_B)],
                            out.at[q, pl.ds((NS - 1) * ROWS_A, ROWS_B)])


def _agg_call(zT, src2, dst2, zrow):
    return pl.kernel(
        _agg_body,
        out_type=jax.ShapeDtypeStruct((4, N, DQ), jnp.float32),
        mesh=_mesh,
        compiler_params=pltpu.CompilerParams(use_tc_tiling_on_sc=False),
        scratch_types=[
            pltpu.VMEM((RPS, CB), jnp.int32),
            pltpu.VMEM((RPS, CB), jnp.int32),
            pltpu.VMEM((ZB, DQ), jnp.float32),
            pltpu.VMEM((CB, DQ), jnp.float32),
            pltpu.VMEM((CB, DQ), jnp.float32),
            pltpu.VMEM((CB, DQ), jnp.float32),
            pltpu.VMEM((CB, DQ), jnp.float32),
            pltpu.VMEM((CB, DQ), jnp.float32),
            pltpu.VMEM((CB, DQ), jnp.float32),
            pltpu.VMEM((CB, DQ), jnp.float32),
            pltpu.VMEM((CB, DQ), jnp.float32),
            pltpu.SemaphoreType.DMA((NB,)),
            pltpu.SemaphoreType.DMA((NB,)),
            pltpu.VMEM_SHARED((N, DQ), jnp.float32),
        ],
    )(zT, src2, dst2, zrow)


_BM = 1000


def _dinv_of(degp_ref):
    return lax.rsqrt(1.0 + degp_ref[0, :, 0:1] + degp_ref[1, :, 0:1])


def _dot(a, b):
    return jnp.dot(a, b, preferred_element_type=jnp.float32,
                   precision=lax.Precision.DEFAULT)


def _split_q(z, o_ref):
    for k in range(4):
        o_ref[k] = z[:, k * DQ:(k + 1) * DQ]


def _mm_scale_body(x_ref, w_ref, degp_ref, o_ref):
    dinv = _dinv_of(degp_ref)
    z = _dot(x_ref[...], w_ref[...]) * dinv
    _split_q(z, o_ref)


def _mm_scale(x, W, degp):
    return pl.pallas_call(
        _mm_scale_body,
        grid=(N // _BM,),
        in_specs=[
            pl.BlockSpec((_BM, H), lambda i: (i, 0)),
            pl.BlockSpec((H, H), lambda i: (0, 0)),
            pl.BlockSpec((2, _BM, 16), lambda i: (0, i, 0)),
        ],
        out_specs=pl.BlockSpec((4, _BM, DQ), lambda i: (0, i, 0)),
        out_shape=jax.ShapeDtypeStruct((4, N, DQ), jnp.float32),
    )(x, W, degp)


def _combine_body(has_res, refs):
    if has_res:
        agg_ref, z_ref, degp_ref, b_ref, res_ref, w_ref, y_ref, zo_ref = refs
    else:
        agg_ref, z_ref, degp_ref, b_ref, w_ref, y_ref, zo_ref = refs
    dinv = _dinv_of(degp_ref)
    aggz = jnp.concatenate(
        [agg_ref[k] + z_ref[k] for k in range(4)], axis=1)
    y = aggz * dinv + b_ref[...]
    if has_res:
        y = y + res_ref[...]
    y = jnp.maximum(y, 0.0)
    y_ref[...] = y
    zn = _dot(y, w_ref[...]) * dinv
    _split_q(zn, zo_ref)


def _combine_mm(agg, z, degp, bias, res, Wn):
    has_res = res is not None
    in_specs = [
        pl.BlockSpec((4, _BM, DQ), lambda i: (0, i, 0)),
        pl.BlockSpec((4, _BM, DQ), lambda i: (0, i, 0)),
        pl.BlockSpec((2, _BM, 16), lambda i: (0, i, 0)),
        pl.BlockSpec((1, H), lambda i: (0, 0)),
    ]
    args = [agg, z, degp, bias]
    if has_res:
        in_specs.append(pl.BlockSpec((_BM, H), lambda i: (i, 0)))
        args.append(res)
    in_specs.append(pl.BlockSpec((H, H), lambda i: (0, 0)))
    args.append(Wn)
    return pl.pallas_call(
        lambda *refs: _combine_body(has_res, refs),
        grid=(N // _BM,),
        in_specs=in_specs,
        out_specs=[
            pl.BlockSpec((_BM, H), lambda i: (i, 0)),
            pl.BlockSpec((4, _BM, DQ), lambda i: (0, i, 0)),
        ],
        out_shape=[
            jax.ShapeDtypeStruct((N, H), jnp.float32),
            jax.ShapeDtypeStruct((4, N, DQ), jnp.float32),
        ],
    )(*args)


def _final_body(agg_ref, z_ref, degp_ref, b_ref, res_ref, w1_ref, b1_ref,
                w2_ref, b2_ref, w3_ref, b3_ref, o_ref):
    dinv = _dinv_of(degp_ref)
    aggz = jnp.concatenate(
        [agg_ref[k] + z_ref[k] for k in range(4)], axis=1)
    y = jnp.maximum(aggz * dinv + b_ref[...] + res_ref[...], 0.0)
    h = jnp.maximum(_dot(y, w1_ref[...]) + b1_ref[...], 0.0)
    h = jnp.maximum(_dot(h, w2_ref[...]) + b2_ref[...], 0.0)
    o = _dot(h, w3_ref[...]) + b3_ref[...]
    m = jnp.max(o, axis=1, keepdims=True)
    e = jnp.exp(o - m)
    o_ref[...] = (o - m) - jnp.log(jnp.sum(e, axis=1, keepdims=True))


def _final(agg, z, degp, bias, res, W1, b1, W2, b2, W3, b3):
    nc = W3.shape[1]
    return pl.pallas_call(
        _final_body,
        grid=(N // _BM,),
        in_specs=[
            pl.BlockSpec((4, _BM, DQ), lambda i: (0, i, 0)),
            pl.BlockSpec((4, _BM, DQ), lambda i: (0, i, 0)),
            pl.BlockSpec((2, _BM, 16), lambda i: (0, i, 0)),
            pl.BlockSpec((1, H), lambda i: (0, 0)),
            pl.BlockSpec((_BM, H), lambda i: (i, 0)),
            pl.BlockSpec((H, H), lambda i: (0, 0)),
            pl.BlockSpec((1, H), lambda i: (0, 0)),
            pl.BlockSpec((H, H), lambda i: (0, 0)),
            pl.BlockSpec((1, H), lambda i: (0, 0)),
            pl.BlockSpec((H, nc), lambda i: (0, 0)),
            pl.BlockSpec((1, nc), lambda i: (0, 0)),
        ],
        out_specs=pl.BlockSpec((_BM, nc), lambda i: (i, 0)),
        out_shape=jax.ShapeDtypeStruct((N, nc), jnp.float32),
    )(agg, z, degp, bias, res, W1, b1, W2, b2, W3, b3)


def kernel(x, edge_index, TRAIN, Wc0, bc0, Wc1, bc1, Wc2, bc2,
           Wf1, bf1, Wf2, bf2, Wf3, bf3):
    del TRAIN
    src2 = edge_index[0].reshape(E // CB, CB)
    dst2 = edge_index[1].reshape(E // CB, CB)
    ones16 = jnp.ones((CB, 16), jnp.float32)
    zer16 = jnp.zeros((DZB, 16), jnp.float32)
    zrow = jnp.zeros((ZB, DQ), jnp.float32)

    degp = _deg_call(dst2, ones16, zer16)
    z0 = _mm_scale(x, Wc0, degp)
    agg0 = _agg_call(z0, src2, dst2, zrow)
    y1, z1 = _combine_mm(agg0, z0, degp, bc0.reshape(1, H), None, Wc1)
    agg1 = _agg_call(z1, src2, dst2, zrow)
    _, z2 = _combine_mm(agg1, z1, degp, bc1.reshape(1, H), y1, Wc2)
    agg2 = _agg_call(z2, src2, dst2, zrow)
    return _final(agg2, z2, degp, bc2.reshape(1, H), y1,
                  Wf1, bf1.reshape(1, H), Wf2, bf2.reshape(1, H),
                  Wf3, bf3.reshape(1, 7))


# speedup vs baseline: 14.2550x; 1.0491x over previous
import jax
import jax.numpy as jnp
from jax import lax
from jax.experimental import pallas as pl
from jax.experimental.pallas import tpu as pltpu
from jax.experimental.pallas import tpu_sc as plsc

N = 10000
E = 160000
H = 256
DQ = 64
NC = 2
NS = 16
ROWS_A = 640
ROWS_B = 400
CB = 125
RPS = 80
RPT = 40
NB = 8
ZB = 80
DZB = 40

_mesh = plsc.VectorSubcoreMesh(core_axis_name="c", subcore_axis_name="s")


def _deg_body(dst2, ones_hbm, zer16_hbm, out, didx, ones_v, zer_v, deg_sh):
    c = lax.axis_index("c")
    s = lax.axis_index("s")
    t = s * NC + c
    pltpu.sync_copy(ones_hbm, ones_v)
    pltpu.sync_copy(zer16_hbm, zer_v)
    r0 = s * ROWS_A
    nz = jnp.where(s == NS - 1, ROWS_B // DZB, ROWS_A // DZB)

    def zb(j, carry):
        pltpu.sync_copy(zer_v, deg_sh.at[pl.ds(r0 + j * DZB, DZB)])
        return carry

    lax.fori_loop(0, nz, zb, 0)
    pltpu.sync_copy(dst2.at[pl.ds(t * RPT, RPT)], didx)
    plsc.subcore_barrier()

    def body(i, carry):
        pltpu.sync_copy(ones_v, deg_sh.at[didx.at[i]], add=True)
        return carry

    lax.fori_loop(0, RPT, body, 0)
    plsc.subcore_barrier()

    @pl.when(s < NS - 1)
    def _():
        pltpu.sync_copy(deg_sh.at[pl.ds(r0, ROWS_A)],
                        out.at[c, pl.ds(r0, ROWS_A)])

    @pl.when(s == NS - 1)
    def _():
        pltpu.sync_copy(deg_sh.at[pl.ds((NS - 1) * ROWS_A, ROWS_B)],
                        out.at[c, pl.ds((NS - 1) * ROWS_A, ROWS_B)])


def _deg_call(dst2, ones16, zer16):
    return pl.kernel(
        _deg_body,
        out_type=jax.ShapeDtypeStruct((NC, N, 16), jnp.float32),
        mesh=_mesh,
        compiler_params=pltpu.CompilerParams(use_tc_tiling_on_sc=False),
        scratch_types=[
            pltpu.VMEM((RPT, CB), jnp.int32),
            pltpu.VMEM((CB, 16), jnp.float32),
            pltpu.VMEM((DZB, 16), jnp.float32),
            pltpu.VMEM_SHARED((N, 16), jnp.float32),
        ],
    )(dst2, ones16, zer16)


def _agg_body(zT, src2, dst2, zrow_hbm, out, sidx, didx, zer_v,
              b0, b1, b2, b3, b4, b5, b6, b7, gsem, ssem, acc_sh):
    c = lax.axis_index("c")
    s = lax.axis_index("s")
    rows = [b0, b1, b2, b3, b4, b5, b6, b7]
    r0 = s * ROWS_A
    pltpu.sync_copy(zrow_hbm, zer_v)
    pltpu.sync_copy(src2.at[pl.ds(s * RPS, RPS)], sidx)
    pltpu.sync_copy(dst2.at[pl.ds(s * RPS, RPS)], didx)
    nz = jnp.where(s == NS - 1, ROWS_B // ZB, ROWS_A // ZB)

    for k in range(2):
        q = 2 * k + c
        zc = zT.at[q]

        def zb(j, carry):
            pltpu.sync_copy(zer_v, acc_sh.at[pl.ds(r0 + j * ZB, ZB)])
            return carry

        lax.fori_loop(0, nz, zb, 0)
        plsc.subcore_barrier()

        for b in range(NB):
            pltpu.async_copy(zc.at[sidx.at[b]], rows[b], gsem.at[b])

        def group(g, carry):
            for b in range(NB):
                i = g * NB + b
                pltpu.make_async_copy(zc.at[sidx.at[i]], rows[b],
                                      gsem.at[b]).wait()
                pltpu.sync_copy(rows[b], acc_sh.at[didx.at[i]], add=True)
                pltpu.async_copy(zc.at[sidx.at[i + NB]], rows[b], gsem.at[b])
            return carry

        lax.fori_loop(0, RPS // NB - 1, group, 0)
        for b in range(NB):
            i = (RPS // NB - 1) * NB + b
            pltpu.make_async_copy(zc.at[sidx.at[i]], rows[b],
                                  gsem.at[b]).wait()
            pltpu.sync_copy(rows[b], acc_sh.at[didx.at[i]], add=True)
        plsc.subcore_barrier()

        @pl.when(s < NS - 1)
        def _():
            pltpu.sync_copy(acc_sh.at[pl.ds(r0, ROWS_A)],
                            out.at[q, pl.ds(r0, ROWS_A)])

        @pl.when(s == NS - 1)
        def _():
            pltpu.sync_copy(acc_sh.at[pl.ds((NS - 1) * ROWS_A, ROWS_B)],
                            out.at[q, pl.ds((NS - 1) * ROWS_A, ROWS_B)])


def _agg_call(zT, src2, dst2, zrow):
    return pl.kernel(
        _agg_body,
        out_type=jax.ShapeDtypeStruct((4, N, DQ), jnp.float32),
        mesh=_mesh,
        compiler_params=pltpu.CompilerParams(use_tc_tiling_on_sc=False),
        scratch_types=[
            pltpu.VMEM((RPS, CB), jnp.int32),
            pltpu.VMEM((RPS, CB), jnp.int32),
            pltpu.VMEM((ZB, DQ), jnp.float32),
            pltpu.VMEM((CB, DQ), jnp.float32),
            pltpu.VMEM((CB, DQ), jnp.float32),
            pltpu.VMEM((CB, DQ), jnp.float32),
            pltpu.VMEM((CB, DQ), jnp.float32),
            pltpu.VMEM((CB, DQ), jnp.float32),
            pltpu.VMEM((CB, DQ), jnp.float32),
            pltpu.VMEM((CB, DQ), jnp.float32),
            pltpu.VMEM((CB, DQ), jnp.float32),
            pltpu.SemaphoreType.DMA((NB,)),
            pltpu.SemaphoreType.DMA((NB,)),
            pltpu.VMEM_SHARED((N, DQ), jnp.float32),
        ],
    )(zT, src2, dst2, zrow)


_BM = 1000


def _dinv_of(degp_ref):
    return lax.rsqrt(1.0 + degp_ref[0, :, 0:1] + degp_ref[1, :, 0:1])


def _dot(a, b):
    return jnp.dot(a, b, preferred_element_type=jnp.float32,
                   precision=lax.Precision.DEFAULT)


def _split_q(z, o_ref):
    for k in range(4):
        o_ref[k] = z[:, k * DQ:(k + 1) * DQ]


def _mm_scale_body(x_ref, w_ref, degp_ref, o_ref):
    dinv = _dinv_of(degp_ref)
    z = _dot(x_ref[...], w_ref[...]) * dinv
    _split_q(z, o_ref)


def _mm_scale(x, W, degp):
    return pl.pallas_call(
        _mm_scale_body,
        grid=(N // _BM,),
        in_specs=[
            pl.BlockSpec((_BM, H), lambda i: (i, 0)),
            pl.BlockSpec((H, H), lambda i: (0, 0)),
            pl.BlockSpec((2, _BM, 16), lambda i: (0, i, 0)),
        ],
        out_specs=pl.BlockSpec((4, _BM, DQ), lambda i: (0, i, 0)),
        out_shape=jax.ShapeDtypeStruct((4, N, DQ), jnp.float32),
    )(x, W, degp)


def _combine_body(has_res, refs):
    if has_res:
        agg_ref, z_ref, degp_ref, b_ref, res_ref, w_ref, y_ref, zo_ref = refs
    else:
        agg_ref, z_ref, degp_ref, b_ref, w_ref, y_ref, zo_ref = refs
    dinv = _dinv_of(degp_ref)
    aggz = jnp.concatenate(
        [agg_ref[k] + z_ref[k] for k in range(4)], axis=1)
    y = aggz * dinv + b_ref[...]
    if has_res:
        y = y + res_ref[...]
    y = jnp.maximum(y, 0.0)
    y_ref[...] = y
    zn = _dot(y, w_ref[...]) * dinv
    _split_q(zn, zo_ref)


def _combine_mm(agg, z, degp, bias, res, Wn):
    has_res = res is not None
    in_specs = [
        pl.BlockSpec((4, _BM, DQ), lambda i: (0, i, 0)),
        pl.BlockSpec((4, _BM, DQ), lambda i: (0, i, 0)),
        pl.BlockSpec((2, _BM, 16), lambda i: (0, i, 0)),
        pl.BlockSpec((1, H), lambda i: (0, 0)),
    ]
    args = [agg, z, degp, bias]
    if has_res:
        in_specs.append(pl.BlockSpec((_BM, H), lambda i: (i, 0)))
        args.append(res)
    in_specs.append(pl.BlockSpec((H, H), lambda i: (0, 0)))
    args.append(Wn)
    return pl.pallas_call(
        lambda *refs: _combine_body(has_res, refs),
        grid=(N // _BM,),
        in_specs=in_specs,
        out_specs=[
            pl.BlockSpec((_BM, H), lambda i: (i, 0)),
            pl.BlockSpec((4, _BM, DQ), lambda i: (0, i, 0)),
        ],
        out_shape=[
            jax.ShapeDtypeStruct((N, H), jnp.float32),
            jax.ShapeDtypeStruct((4, N, DQ), jnp.float32),
        ],
    )(*args)


def _final_body(agg_ref, z_ref, degp_ref, b_ref, res_ref, w1_ref, b1_ref,
                w2_ref, b2_ref, w3_ref, b3_ref, o_ref):
    dinv = _dinv_of(degp_ref)
    aggz = jnp.concatenate(
        [agg_ref[k] + z_ref[k] for k in range(4)], axis=1)
    y = jnp.maximum(aggz * dinv + b_ref[...] + res_ref[...], 0.0)
    h = jnp.maximum(_dot(y, w1_ref[...]) + b1_ref[...], 0.0)
    h = jnp.maximum(_dot(h, w2_ref[...]) + b2_ref[...], 0.0)
    o = _dot(h, w3_ref[...]) + b3_ref[...]
    m = jnp.max(o, axis=1, keepdims=True)
    e = jnp.exp(o - m)
    o_ref[...] = (o - m) - jnp.log(jnp.sum(e, axis=1, keepdims=True))


def _final(agg, z, degp, bias, res, W1, b1, W2, b2, W3, b3):
    nc = W3.shape[1]
    return pl.pallas_call(
        _final_body,
        grid=(N // _BM,),
        in_specs=[
            pl.BlockSpec((4, _BM, DQ), lambda i: (0, i, 0)),
            pl.BlockSpec((4, _BM, DQ), lambda i: (0, i, 0)),
            pl.BlockSpec((2, _BM, 16), lambda i: (0, i, 0)),
            pl.BlockSpec((1, H), lambda i: (0, 0)),
            pl.BlockSpec((_BM, H), lambda i: (i, 0)),
            pl.BlockSpec((H, H), lambda i: (0, 0)),
            pl.BlockSpec((1, H), lambda i: (0, 0)),
            pl.BlockSpec((H, H), lambda i: (0, 0)),
            pl.BlockSpec((1, H), lambda i: (0, 0)),
            pl.BlockSpec((H, nc), lambda i: (0, 0)),
            pl.BlockSpec((1, nc), lambda i: (0, 0)),
        ],
        out_specs=pl.BlockSpec((_BM, nc), lambda i: (i, 0)),
        out_shape=jax.ShapeDtypeStruct((N, nc), jnp.float32),
    )(agg, z, degp, bias, res, W1, b1, W2, b2, W3, b3)


def kernel(x, edge_index, TRAIN, Wc0, bc0, Wc1, bc1, Wc2, bc2,
           Wf1, bf1, Wf2, bf2, Wf3, bf3):
    del TRAIN
    src2 = edge_index[0].reshape(E // CB, CB)
    dst2 = edge_index[1].reshape(E // CB, CB)
    ones16 = jnp.ones((CB, 16), jnp.float32)
    zer16 = jnp.zeros((DZB, 16), jnp.float32)
    zrow = jnp.zeros((ZB, DQ), jnp.float32)

    degp = _deg_call(dst2, ones16, zer16)
    z0 = _mm_scale(x, Wc0, degp)
    agg0 = _agg_call(z0, src2, dst2, zrow)
    y1, z1 = _combine_mm(agg0, z0, degp, bc0.reshape(1, H), None, Wc1)
    agg1 = _agg_call(z1, src2, dst2, zrow)
    _, z2 = _combine_mm(agg1, z1, degp, bc1.reshape(1, H), y1, Wc2)
    agg2 = _agg_call(z2, src2, dst2, zrow)
    return _final(agg2, z2, degp, bc2.reshape(1, H), y1,
                  Wf1, bf1.reshape(1, H), Wf2, bf2.reshape(1, H),
                  Wf3, bf3.reshape(1, 7))
